# baseline (speedup 1.0000x reference)
import jax
import jax.numpy as jnp
from jax import lax
from jax.experimental import pallas as pl
from jax.experimental.pallas import tpu as pltpu
from jax.experimental.pallas import tpu_sc as plsc

X, Y, H = 256, 256, 16
N_COARSE = 16384
BT = 8
NC, NS = 2, 16
NW = NC * NS
NHALF = 2
PER_H = N_COARSE // NHALF // NW
CHUNK = 128
NCHUNK = PER_H // CHUNK
SPAN = PER_H * H
NJ = N_COARSE * H
NJH = NJ // NHALF

SCALE_XY = 102.4
DIST_XY = 51.2
SCALE_H = 8.0
DIST_H = 5.0

_GDN = lax.GatherDimensionNumbers(
    offset_dims=(), collapsed_slice_dims=(0,), start_index_map=(0,))


def _splat(vec, k):
    idx = jnp.full((16, 1), k, jnp.int32)
    return lax.gather(vec, idx, dimension_numbers=_GDN, slice_sizes=(1,),
                      mode=lax.GatherScatterMode.PROMISE_IN_BOUNDS)


def _make_sc_body(half):
    def _sc_body(xtab, ytab, rnd, xexp_out, yexp_out,
                 idx_v, rows_x, rows_y, xe, ye, gsem, osem):
        wid = lax.axis_index("s") * NC + lax.axis_index("c")
        base = half * (N_COARSE // NHALF) + wid * PER_H
        pltpu.sync_copy(rnd.at[pl.ds(base, PER_H)], idx_v)
        gathers = []
        for k in range(NCHUNK):
            sl = pl.ds(k * CHUNK, CHUNK)
            gathers.append(pltpu.async_copy(
                xtab.at[idx_v.at[sl]], rows_x.at[sl], gsem))
            gathers.append(pltpu.async_copy(
                ytab.at[idx_v.at[sl]], rows_y.at[sl], gsem))
        for g in gathers:
            g.wait()

        def body(i, carry):
            xv = rows_x[pl.ds(i * 16, 16)]
            yv = rows_y[pl.ds(i * 16, 16)]
            for k in range(16):
                sl = pl.ds((i * 16 + k) * H, H)
                xe[sl] = _splat(xv, k)
                ye[sl] = _splat(yv, k)
            return carry

        lax.fori_loop(0, PER_H // 16, body, 0)

        obase = wid * SPAN
        c1 = pltpu.async_copy(xe, xexp_out.at[0, pl.ds(obase, SPAN)], osem)
        c2 = pltpu.async_copy(ye, yexp_out.at[0, pl.ds(obase, SPAN)], osem)
        c1.wait()
        c2.wait()

    return _sc_body


TCW = NJ // 4


def _tc_heights(btzf_ref, btzi_ref, coords_ref, idx_ref):
    btzf = btzf_ref[0, 0]
    btzi = btzi_ref[0, 0]
    h = lax.broadcasted_iota(jnp.int32, (1, TCW), 1) & (H - 1)
    hf = h.astype(jnp.float32) * (1.0 / (H - 1))
    ch = hf * SCALE_H - DIST_H + btzf
    ih = h + btzi
    coords_ref[0] = jnp.broadcast_to(ch, (BT, TCW))
    idx_ref[0] = jnp.broadcast_to(ih, (BT, TCW))


TCWH = NJH // 4


def _tc_xy(xexp_ref, yexp_ref, btzf_ref, btzi_ref, c_in_ref, i_in_ref,
           coords_ref, idx_ref):
    del c_in_ref, i_in_ref
    btzf = btzf_ref[0, 0]
    btzi = btzi_ref[0, 0]
    xv = xexp_ref[...]
    yv = yexp_ref[...]
    cx = xv * SCALE_XY - DIST_XY + btzf
    cy = yv * SCALE_XY - DIST_XY + btzf
    ix = (xv * float(X - 1) + 0.5).astype(jnp.int32) + btzi
    iy = (yv * float(Y - 1) + 0.5).astype(jnp.int32) + btzi
    coords_ref[0] = jnp.broadcast_to(cx, (BT, TCWH))
    coords_ref[1] = jnp.broadcast_to(cy, (BT, TCWH))
    idx_ref[0] = jnp.broadcast_to(ix, (BT, TCWH))
    idx_ref[1] = jnp.broadcast_to(iy, (BT, TCWH))


def kernel(grid, rnd, bt):
    xtab = grid[:, :, 0].reshape(X * Y)
    ytab = grid[:, :, 1].reshape(X * Y)
    btz = (jnp.asarray(bt) - BT).astype(jnp.int32)
    btzi = btz.reshape(1, 1)
    btzf = btzi.astype(jnp.float32)

    mesh = plsc.VectorSubcoreMesh(
        core_axis_name="c", subcore_axis_name="s",
        num_cores=NC, num_subcores=NS)
    sc_scratch = [
        pltpu.VMEM((PER_H,), jnp.int32),
        pltpu.VMEM((PER_H,), jnp.float32),
        pltpu.VMEM((PER_H,), jnp.float32),
        pltpu.VMEM((SPAN,), jnp.float32),
        pltpu.VMEM((SPAN,), jnp.float32),
        pltpu.SemaphoreType.DMA,
        pltpu.SemaphoreType.DMA,
    ]
    sc_out = (
        jax.ShapeDtypeStruct((1, NJH), jnp.float32),
        jax.ShapeDtypeStruct((1, NJH), jnp.float32),
    )
    halves = []
    for h in range(NHALF):
        run = pl.kernel(_make_sc_body(h), out_type=sc_out, mesh=mesh,
                        scratch_types=sc_scratch)
        halves.append(run(xtab, ytab, rnd))

    coords, vidx = pl.pallas_call(
        _tc_heights,
        grid=(NJ // TCW,),
        in_specs=[
            pl.BlockSpec(memory_space=pltpu.SMEM),
            pl.BlockSpec(memory_space=pltpu.SMEM),
        ],
        out_specs=[
            pl.BlockSpec((1, BT, TCW), lambda n: (2, 0, n)),
            pl.BlockSpec((1, BT, TCW), lambda n: (2, 0, n)),
        ],
        out_shape=(
            jax.ShapeDtypeStruct((3, BT, NJ), jnp.float32),
            jax.ShapeDtypeStruct((3, BT, NJ), jnp.int32),
        ),
    )(btzf, btzi)

    nsteps = NJH // TCWH
    for h in range(NHALF):
        xexp, yexp = halves[h]
        coords, vidx = pl.pallas_call(
            _tc_xy,
            grid=(nsteps,),
            in_specs=[
                pl.BlockSpec((1, TCWH), lambda n: (0, n)),
                pl.BlockSpec((1, TCWH), lambda n: (0, n)),
                pl.BlockSpec(memory_space=pltpu.SMEM),
                pl.BlockSpec(memory_space=pltpu.SMEM),
                pl.BlockSpec(memory_space=pl.ANY),
                pl.BlockSpec(memory_space=pl.ANY),
            ],
            out_specs=[
                pl.BlockSpec((2, BT, TCWH),
                             lambda n, h=h: (0, 0, h * nsteps + n)),
                pl.BlockSpec((2, BT, TCWH),
                             lambda n, h=h: (0, 0, h * nsteps + n)),
            ],
            out_shape=(
                jax.ShapeDtypeStruct((3, BT, NJ), jnp.float32),
                jax.ShapeDtypeStruct((3, BT, NJ), jnp.int32),
            ),
            input_output_aliases={4: 0, 5: 1},
        )(xexp, yexp, btzf, btzi, coords, vidx)
    return (coords.transpose(1, 0, 2), vidx.transpose(1, 0, 2))


# speedup vs baseline: 1.0733x; 1.0733x over previous
import jax
import jax.numpy as jnp
from jax import lax
from jax.experimental import pallas as pl
from jax.experimental.pallas import tpu as pltpu
from jax.experimental.pallas import tpu_sc as plsc

X, Y, H = 256, 256, 16
N_COARSE = 16384
BT = 8
NC, NS = 2, 16
NW = NC * NS
PER_W = N_COARSE // NW
CHUNK = 128
NCHUNK = PER_W // CHUNK
SPAN = PER_W * H
NJ = N_COARSE * H

SCALE_XY = 102.4
DIST_XY = 51.2
SCALE_H = 8.0
DIST_H = 5.0

_GDN = lax.GatherDimensionNumbers(
    offset_dims=(), collapsed_slice_dims=(0,), start_index_map=(0,))


def _splat(vec, k):
    idx = jnp.full((16, 1), k, jnp.int32)
    return lax.gather(vec, idx, dimension_numbers=_GDN, slice_sizes=(1,),
                      mode=lax.GatherScatterMode.PROMISE_IN_BOUNDS)


def _sc_body(xtab, ytab, rnd, xexp_out, yexp_out,
             idx_v, rows_x, rows_y, xe, ye, gsem, osem):
    wid = lax.axis_index("s") * NC + lax.axis_index("c")
    base = wid * PER_W
    pltpu.sync_copy(rnd.at[pl.ds(base, PER_W)], idx_v)
    gathers = []
    for k in range(NCHUNK):
        sl = pl.ds(k * CHUNK, CHUNK)
        gathers.append(pltpu.async_copy(
            xtab.at[idx_v.at[sl]], rows_x.at[sl], gsem))
        gathers.append(pltpu.async_copy(
            ytab.at[idx_v.at[sl]], rows_y.at[sl], gsem))
    for g in gathers:
        g.wait()

    def body(i, carry):
        xv = rows_x[pl.ds(i * 16, 16)]
        yv = rows_y[pl.ds(i * 16, 16)]
        for k in range(16):
            sl = pl.ds((i * 16 + k) * H, H)
            xe[sl] = _splat(xv, k)
            ye[sl] = _splat(yv, k)
        return carry

    lax.fori_loop(0, PER_W // 16, body, 0)

    c1 = pltpu.async_copy(xe, xexp_out.at[0, pl.ds(base * H, SPAN)], osem)
    c2 = pltpu.async_copy(ye, yexp_out.at[0, pl.ds(base * H, SPAN)], osem)
    c1.wait()
    c2.wait()


TCW = NJ // 4


def _tc_heights(btzf_ref, btzi_ref, coords_ref, idx_ref):
    btzf = btzf_ref[0, 0]
    btzi = btzi_ref[0, 0]
    h = lax.broadcasted_iota(jnp.int32, (1, TCW), 1) & (H - 1)
    hf = h.astype(jnp.float32) * (1.0 / (H - 1))
    ch = hf * SCALE_H - DIST_H + btzf
    ih = h + btzi
    coords_ref[0] = jnp.broadcast_to(ch, (BT, TCW))
    idx_ref[0] = jnp.broadcast_to(ih, (BT, TCW))


def _tc_xy(xexp_ref, yexp_ref, btzf_ref, btzi_ref, c_in_ref, i_in_ref,
           coords_ref, idx_ref):
    del c_in_ref, i_in_ref
    btzf = btzf_ref[0, 0]
    btzi = btzi_ref[0, 0]
    xv = xexp_ref[...]
    yv = yexp_ref[...]
    cx = xv * SCALE_XY - DIST_XY + btzf
    cy = yv * SCALE_XY - DIST_XY + btzf
    ix = (xv * float(X - 1) + 0.5).astype(jnp.int32) + btzi
    iy = (yv * float(Y - 1) + 0.5).astype(jnp.int32) + btzi
    coords_ref[0] = jnp.broadcast_to(cx, (BT, TCW))
    coords_ref[1] = jnp.broadcast_to(cy, (BT, TCW))
    idx_ref[0] = jnp.broadcast_to(ix, (BT, TCW))
    idx_ref[1] = jnp.broadcast_to(iy, (BT, TCW))


def kernel(grid, rnd, bt):
    xtab = grid[:, :, 0].reshape(X * Y)
    ytab = grid[:, :, 1].reshape(X * Y)
    btz = (jnp.asarray(bt) - BT).astype(jnp.int32)
    btzi = btz.reshape(1, 1)
    btzf = btzi.astype(jnp.float32)

    mesh = plsc.VectorSubcoreMesh(
        core_axis_name="c", subcore_axis_name="s",
        num_cores=NC, num_subcores=NS)
    sc_run = pl.kernel(
        _sc_body,
        out_type=(
            jax.ShapeDtypeStruct((1, NJ), jnp.float32),
            jax.ShapeDtypeStruct((1, NJ), jnp.float32),
        ),
        mesh=mesh,
        scratch_types=[
            pltpu.VMEM((PER_W,), jnp.int32),
            pltpu.VMEM((PER_W,), jnp.float32),
            pltpu.VMEM((PER_W,), jnp.float32),
            pltpu.VMEM((SPAN,), jnp.float32),
            pltpu.VMEM((SPAN,), jnp.float32),
            pltpu.SemaphoreType.DMA,
            pltpu.SemaphoreType.DMA,
        ],
    )
    xexp, yexp = sc_run(xtab, ytab, rnd)

    coords_h, vidx_h = pl.pallas_call(
        _tc_heights,
        grid=(NJ // TCW,),
        in_specs=[
            pl.BlockSpec(memory_space=pltpu.SMEM),
            pl.BlockSpec(memory_space=pltpu.SMEM),
        ],
        out_specs=[
            pl.BlockSpec((1, BT, TCW), lambda n: (2, 0, n)),
            pl.BlockSpec((1, BT, TCW), lambda n: (2, 0, n)),
        ],
        out_shape=(
            jax.ShapeDtypeStruct((3, BT, NJ), jnp.float32),
            jax.ShapeDtypeStruct((3, BT, NJ), jnp.int32),
        ),
    )(btzf, btzi)

    coords, vidx = pl.pallas_call(
        _tc_xy,
        grid=(NJ // TCW,),
        in_specs=[
            pl.BlockSpec((1, TCW), lambda n: (0, n)),
            pl.BlockSpec((1, TCW), lambda n: (0, n)),
            pl.BlockSpec(memory_space=pltpu.SMEM),
            pl.BlockSpec(memory_space=pltpu.SMEM),
            pl.BlockSpec(memory_space=pl.ANY),
            pl.BlockSpec(memory_space=pl.ANY),
        ],
        out_specs=[
            pl.BlockSpec((2, BT, TCW), lambda n: (0, 0, n)),
            pl.BlockSpec((2, BT, TCW), lambda n: (0, 0, n)),
        ],
        out_shape=(
            jax.ShapeDtypeStruct((3, BT, NJ), jnp.float32),
            jax.ShapeDtypeStruct((3, BT, NJ), jnp.int32),
        ),
        input_output_aliases={4: 0, 5: 1},
    )(xexp, yexp, btzf, btzi, coords_h, vidx_h)
    return (coords.transpose(1, 0, 2), vidx.transpose(1, 0, 2))
